# baseline (speedup 1.0000x reference)
import functools
import jax
import jax.numpy as jnp
from jax import lax
from jax.experimental import pallas as pl
from jax.experimental.pallas import tpu as pltpu
from jax.experimental.pallas import tpu_sc as plsc

_CH = 32


def _make_sc(batch, seq_len, d_model, dtype):
    info = plsc.get_sparse_core_info()
    nc, ns = info.num_cores, info.num_subcores
    nw = nc * ns
    rows_per_w = seq_len // nw
    nchunks = rows_per_w // _CH
    mesh = plsc.VectorSubcoreMesh(core_axis_name="c", subcore_axis_name="s")

    @functools.partial(
        pl.kernel,
        mesh=mesh,
        out_type=jax.ShapeDtypeStruct((batch * seq_len, d_model), dtype),
        scratch_types=[
            pltpu.VMEM((2, _CH, d_model), dtype),
            pltpu.SemaphoreType.DMA((2,)),
            pltpu.SemaphoreType.DMA((2,)),
        ],
    )
    def k(w_hbm, out_hbm, buf, insem, outsem):
        wid = lax.axis_index("s") * nc + lax.axis_index("c")
        base = wid * rows_per_w

        def in_copy(j, slot):
            r = base + j * _CH
            return pltpu.make_async_copy(
                w_hbm.at[pl.ds(r, _CH), :], buf.at[slot], insem.at[slot]
            )

        def out_copy(j, slot, b):
            r = base + j * _CH
            return pltpu.make_async_copy(
                buf.at[slot],
                out_hbm.at[pl.ds(b * seq_len + r, _CH), :],
                outsem.at[slot],
            )

        in_copy(0, 0).start()
        for j in range(nchunks):
            slot = j % 2
            if j + 1 < nchunks:
                if j >= 1:
                    for b in range(batch):
                        out_copy(j - 1, (j - 1) % 2, b).wait()
                in_copy(j + 1, (j + 1) % 2).start()
            in_copy(j, slot).wait()
            for b in range(batch):
                out_copy(j, slot, b).start()
        for j in (nchunks - 2, nchunks - 1):
            for b in range(batch):
                out_copy(j, j % 2, b).wait()

    return k


def kernel(tokens, W_pos):
    batch, seq_len = tokens.shape
    d_model = W_pos.shape[1]
    flat = _make_sc(batch, seq_len, d_model, W_pos.dtype)(W_pos)
    return flat.reshape(batch, seq_len, d_model)
